# baseline (speedup 1.0000x reference)
import functools
import math

import jax
import jax.numpy as jnp
from jax import lax
from jax.experimental import pallas as pl
from jax.experimental.pallas import tpu as pltpu
from jax.experimental.pallas import tpu_sc as plsc

B = 4096
P = 8192
D = 32
TEMP = 0.5
MOM = 0.9
LN_M = math.log(MOM)

RB = 8
BR = B // RB
CB = 8
PC = P // CB

NC = 2
NS = 16
TROWS = P // (NC * NS)
WCHUNK = 256


def _weights_body(cidr_ref, cidc_ref, emb_ref, we_ref):
    r = pl.program_id(0)
    cidr = cidr_ref[...]
    row_gid = r * BR + lax.broadcasted_iota(jnp.int32, (BR, 1), 0)

    def step(k, acc):
        cidc = cidc_ref[:, pl.ds(k * 1024, 1024)]
        col_gid = k * 1024 + lax.broadcasted_iota(jnp.int32, (1, 1024), 1)
        hit = (cidr == cidc) & (col_gid > row_gid)
        return acc + jnp.sum(jnp.where(hit, 1.0, 0.0), axis=1, keepdims=True)

    occ_after = lax.fori_loop(0, B // 1024, step, jnp.zeros((BR, 1), jnp.float32))
    w = (1.0 - MOM) * jnp.exp(occ_after * LN_M)
    we_ref[...] = emb_ref[...] * w


def _loss_body(emb_ref, proto_ref, cidcol_ref, cidrow_ref,
               loss_ref, decayed_ref,
               sumexp_ref, pos_ref, cnt_ref, lossacc_ref):
    r = pl.program_id(0)
    c = pl.program_id(1)

    emb = emb_ref[...]
    en = emb * lax.rsqrt(jnp.maximum(jnp.sum(emb * emb, axis=1, keepdims=True), 1e-24))
    pr = proto_ref[...]
    pn = pr * lax.rsqrt(jnp.maximum(jnp.sum(pr * pr, axis=1, keepdims=True), 1e-24))
    s = lax.dot_general(en, pn, (((1,), (1,)), ((), ())),
                        preferred_element_type=jnp.float32) * (1.0 / TEMP)

    cid_col = cidcol_ref[...]
    col_gid = c * PC + lax.broadcasted_iota(jnp.int32, (1, PC), 1)
    is_pos = cid_col == col_gid

    prev_se = jnp.where(c == 0, jnp.zeros((BR, 1), jnp.float32), sumexp_ref[...])
    sumexp_ref[...] = prev_se + jnp.sum(jnp.exp(s), axis=1, keepdims=True)
    prev_pos = jnp.where(c == 0, jnp.zeros((BR, 1), jnp.float32), pos_ref[...])
    pos_ref[...] = prev_pos + jnp.sum(jnp.where(is_pos, s, 0.0), axis=1, keepdims=True)

    cid_row = cidrow_ref[...]
    colv = c * PC + lax.broadcasted_iota(jnp.int32, (PC, 1), 0)
    hits = colv == cid_row
    contrib = jnp.sum(jnp.where(hits, 1.0, 0.0), axis=1, keepdims=True)
    prev_cnt = jnp.where(r == 0, jnp.zeros((PC, 1), jnp.float32),
                         cnt_ref[pl.ds(c * PC, PC), :])
    cnt = prev_cnt + contrib
    cnt_ref[pl.ds(c * PC, PC), :] = cnt

    decayed_ref[...] = pr * jnp.exp(cnt * LN_M)

    @pl.when(c == CB - 1)
    def _finish_row_block():
        pos = pos_ref[...]
        se = sumexp_ref[...]
        row_loss = -pos + jnp.log(se - jnp.exp(pos))
        prev = jnp.where(r == 0, jnp.zeros((1, 1), jnp.float32), lossacc_ref[...])
        lossacc_ref[...] = prev + jnp.sum(row_loss, axis=(0, 1), keepdims=True)

    @pl.when((c == CB - 1) & (r == RB - 1))
    def _emit_loss():
        loss_ref[...] = lossacc_ref[...] * (1.0 / B)


_weights_call = pl.pallas_call(
    _weights_body,
    grid=(RB,),
    in_specs=[
        pl.BlockSpec((BR, 1), lambda r: (r, 0)),
        pl.BlockSpec((1, B), lambda r: (0, 0)),
        pl.BlockSpec((BR, D), lambda r: (r, 0)),
    ],
    out_specs=pl.BlockSpec((BR, D), lambda r: (r, 0)),
    out_shape=jax.ShapeDtypeStruct((B, D), jnp.float32),
)

_loss_call = pl.pallas_call(
    _loss_body,
    grid=(RB, CB),
    in_specs=[
        pl.BlockSpec((BR, D), lambda r, c: (r, 0)),
        pl.BlockSpec((PC, D), lambda r, c: (c, 0)),
        pl.BlockSpec((BR, 1), lambda r, c: (r, 0)),
        pl.BlockSpec((1, BR), lambda r, c: (0, r)),
    ],
    out_specs=[
        pl.BlockSpec((1, 1), lambda r, c: (0, 0)),
        pl.BlockSpec((PC, D), lambda r, c: (c, 0)),
    ],
    out_shape=[
        jax.ShapeDtypeStruct((1, 1), jnp.float32),
        jax.ShapeDtypeStruct((P, D), jnp.float32),
    ],
    scratch_shapes=[
        pltpu.VMEM((BR, 1), jnp.float32),
        pltpu.VMEM((BR, 1), jnp.float32),
        pltpu.VMEM((P, 1), jnp.float32),
        pltpu.VMEM((1, 1), jnp.float32),
    ],
)


def _scatter_body(cidrow_ref, we_ref, decayed_ref, out_ref):
    c = pl.program_id(0)
    colv = c * PC + lax.broadcasted_iota(jnp.int32, (PC, 1), 0)
    acc = decayed_ref[...]
    for ch in range(B // 1024):
        cid_row = cidrow_ref[:, pl.ds(ch * 1024, 1024)]
        onehot = jnp.where(colv == cid_row, 1.0, 0.0)
        wec = we_ref[pl.ds(ch * 1024, 1024), :]
        acc = acc + lax.dot_general(onehot, wec, (((1,), (0,)), ((), ())),
                                    preferred_element_type=jnp.float32)
    out_ref[...] = acc


_scatter_call = pl.pallas_call(
    _scatter_body,
    grid=(CB,),
    in_specs=[
        pl.BlockSpec((1, B), lambda c: (0, 0)),
        pl.BlockSpec((B, D), lambda c: (0, 0)),
        pl.BlockSpec((PC, D), lambda c: (c, 0)),
    ],
    out_specs=pl.BlockSpec((PC, D), lambda c: (c, 0)),
    out_shape=jax.ShapeDtypeStruct((P, D), jnp.float32),
)


@functools.cache
def _make_sc_update():
    return pl.kernel(
        _sc_update_body,
        out_type=jax.ShapeDtypeStruct((P, D), jnp.float32),
        mesh=plsc.VectorSubcoreMesh(core_axis_name="c", subcore_axis_name="s",
                                    num_cores=NC, num_subcores=NS),
        scratch_types=[
            pltpu.VMEM((TROWS, D), jnp.float32),
            pltpu.VMEM((B + 16,), jnp.int32),
            pltpu.VMEM((WCHUNK, D), jnp.float32),
        ],
    )


def _sc_update_body(decayed_hbm, cid_hbm, we_hbm, out_hbm,
                    proto_v, ids_v, we_v):
    cidx = lax.axis_index("c")
    sidx = lax.axis_index("s")
    gid = cidx * NS + sidx
    base = gid * TROWS

    pltpu.sync_copy(decayed_hbm.at[pl.ds(base, TROWS)], proto_v)
    pltpu.sync_copy(cid_hbm, ids_v.at[pl.ds(0, B)])

    def _chunk(ch, carry):
        pltpu.sync_copy(we_hbm.at[pl.ds(ch * WCHUNK, WCHUNK)], we_v)

        def _sample(i, c3):
            loc = ids_v[pl.ds(ch * WCHUNK + i, 16)][0] - base

            @pl.when((loc >= 0) & (loc < TROWS))
            def _():
                proto_v[loc, pl.ds(0, 16)] = (
                    proto_v[loc, pl.ds(0, 16)] + we_v[i, pl.ds(0, 16)])
                proto_v[loc, pl.ds(16, 16)] = (
                    proto_v[loc, pl.ds(16, 16)] + we_v[i, pl.ds(16, 16)])

            return c3

        lax.fori_loop(0, WCHUNK, _sample, 0)
        return carry

    lax.fori_loop(0, B // WCHUNK, _chunk, 0)

    pltpu.sync_copy(proto_v, out_hbm.at[pl.ds(base, TROWS)])


def kernel(embeddings, cluster_ids, prototypes):
    cid_col = cluster_ids.reshape(B, 1)
    cid_row = cluster_ids.reshape(1, B)
    we = _weights_call(cid_col, cid_row, embeddings)
    loss2d, decayed = _loss_call(embeddings, prototypes, cid_col, cid_row)
    new_protos = _make_sc_update()(decayed, cluster_ids, we)
    return loss2d[0, 0], new_protos


# speedup vs baseline: 1.7522x; 1.7522x over previous
import functools
import math

import jax
import jax.numpy as jnp
from jax import lax
from jax.experimental import pallas as pl
from jax.experimental.pallas import tpu as pltpu
from jax.experimental.pallas import tpu_sc as plsc

B = 4096
P = 8192
D = 32
TEMP = 0.5
MOM = 0.9
LN_M = math.log(MOM)

RB = 8
BR = B // RB
CB = 8
PC = P // CB

NC = 2
NS = 16
NTILES = NC * NS
TROWS = P // NTILES
TSHIFT = 8
WCHUNK = 256
NGRP = B // 16


def _weights_body(cidr_ref, cidc_ref, emb_ref, proto_ref,
                  we_ref, decayed_ref, hitcnt_ref):
    r = pl.program_id(0)
    cidr = cidr_ref[...]
    row_gid = r * BR + lax.broadcasted_iota(jnp.int32, (BR, 1), 0)

    def step(k, acc):
        cidc = cidc_ref[:, pl.ds(k * 1024, 1024)]
        col_gid = k * 1024 + lax.broadcasted_iota(jnp.int32, (1, 1024), 1)
        hit = (cidr == cidc) & (col_gid > row_gid)
        return acc + jnp.sum(jnp.where(hit, 1.0, 0.0), axis=1, keepdims=True)

    occ_after = lax.fori_loop(0, B // 1024, step, jnp.zeros((BR, 1), jnp.float32))
    w = (1.0 - MOM) * jnp.exp(occ_after * LN_M)
    we_ref[...] = emb_ref[...] * w

    colv = r * PC + lax.broadcasted_iota(jnp.int32, (PC, 1), 0)

    def cstep(k, acc):
        cidc = cidc_ref[:, pl.ds(k * 1024, 1024)]
        return acc + jnp.sum(jnp.where(colv == cidc, 1.0, 0.0),
                             axis=1, keepdims=True)

    cnt = lax.fori_loop(0, B // 1024, cstep, jnp.zeros((PC, 1), jnp.float32))
    decayed_ref[...] = proto_ref[...] * jnp.exp(cnt * LN_M)

    cids = cidc_ref[:, pl.ds(r * BR, BR)]
    tilev = lax.broadcasted_iota(jnp.int32, (NTILES, 1), 0)
    teq = jnp.where(tilev == lax.shift_right_logical(cids, TSHIFT), 1.0, 0.0)
    rowv = lax.broadcasted_iota(jnp.int32, (BR, 1), 0) // 16
    gsel = jnp.where(rowv == lax.broadcasted_iota(jnp.int32, (1, BR // 16), 1),
                     1.0, 0.0)
    hc = lax.dot_general(teq, gsel, (((1,), (0,)), ((), ())),
                         preferred_element_type=jnp.float32)
    hitcnt_ref[...] = hc.reshape(1, NTILES, BR // 16)


def _loss_body(emb_ref, proto_ref, cidcol_ref,
               loss_ref,
               sumexp_ref, pos_ref, lossacc_ref):
    r = pl.program_id(0)
    c = pl.program_id(1)

    emb = emb_ref[...]
    en = emb * lax.rsqrt(jnp.maximum(jnp.sum(emb * emb, axis=1, keepdims=True), 1e-24))
    pr = proto_ref[...]
    pn = pr * lax.rsqrt(jnp.maximum(jnp.sum(pr * pr, axis=1, keepdims=True), 1e-24))
    s = lax.dot_general(en, pn, (((1,), (1,)), ((), ())),
                        preferred_element_type=jnp.float32) * (1.0 / TEMP)

    cid_col = cidcol_ref[...]
    col_gid = c * PC + lax.broadcasted_iota(jnp.int32, (1, PC), 1)
    is_pos = cid_col == col_gid

    prev_se = jnp.where(c == 0, jnp.zeros((BR, 1), jnp.float32), sumexp_ref[...])
    sumexp_ref[...] = prev_se + jnp.sum(jnp.exp(s), axis=1, keepdims=True)
    prev_pos = jnp.where(c == 0, jnp.zeros((BR, 1), jnp.float32), pos_ref[...])
    pos_ref[...] = prev_pos + jnp.sum(jnp.where(is_pos, s, 0.0), axis=1, keepdims=True)

    @pl.when(c == CB - 1)
    def _finish_row_block():
        pos = pos_ref[...]
        se = sumexp_ref[...]
        row_loss = -pos + jnp.log(se - jnp.exp(pos))
        prev = jnp.where(r == 0, jnp.zeros((1, 1), jnp.float32), lossacc_ref[...])
        lossacc_ref[...] = prev + jnp.sum(row_loss, axis=(0, 1), keepdims=True)

    @pl.when((c == CB - 1) & (r == RB - 1))
    def _emit_loss():
        loss_ref[...] = lossacc_ref[...] * (1.0 / B)


_weights_call = pl.pallas_call(
    _weights_body,
    grid=(RB,),
    in_specs=[
        pl.BlockSpec((BR, 1), lambda r: (r, 0)),
        pl.BlockSpec((1, B), lambda r: (0, 0)),
        pl.BlockSpec((BR, D), lambda r: (r, 0)),
        pl.BlockSpec((PC, D), lambda r: (r, 0)),
    ],
    out_specs=[
        pl.BlockSpec((BR, D), lambda r: (r, 0)),
        pl.BlockSpec((PC, D), lambda r: (r, 0)),
        pl.BlockSpec((1, NTILES, BR // 16), lambda r: (r, 0, 0)),
    ],
    out_shape=[
        jax.ShapeDtypeStruct((B, D), jnp.float32),
        jax.ShapeDtypeStruct((P, D), jnp.float32),
        jax.ShapeDtypeStruct((RB, NTILES, BR // 16), jnp.float32),
    ],
)

_loss_call = pl.pallas_call(
    _loss_body,
    grid=(RB, CB),
    in_specs=[
        pl.BlockSpec((BR, D), lambda r, c: (r, 0)),
        pl.BlockSpec((PC, D), lambda r, c: (c, 0)),
        pl.BlockSpec((BR, 1), lambda r, c: (r, 0)),
    ],
    out_specs=pl.BlockSpec((1, 1), lambda r, c: (0, 0)),
    out_shape=jax.ShapeDtypeStruct((1, 1), jnp.float32),
    scratch_shapes=[
        pltpu.VMEM((BR, 1), jnp.float32),
        pltpu.VMEM((BR, 1), jnp.float32),
        pltpu.VMEM((1, 1), jnp.float32),
    ],
)


@functools.cache
def _make_sc_update():
    return pl.kernel(
        _sc_update_body,
        out_type=jax.ShapeDtypeStruct((P, D), jnp.float32),
        mesh=plsc.VectorSubcoreMesh(core_axis_name="c", subcore_axis_name="s",
                                    num_cores=NC, num_subcores=NS),
        scratch_types=[
            pltpu.VMEM((TROWS, D), jnp.float32),
            pltpu.VMEM((B + 16,), jnp.int32),
            pltpu.VMEM((WCHUNK, D), jnp.float32),
            pltpu.VMEM((NGRP + 16,), jnp.float32),
        ],
    )


def _sc_update_body(decayed_hbm, cid_hbm, we_hbm, hitcnt_hbm, out_hbm,
                    proto_v, ids_v, we_v, hits_v):
    cidx = lax.axis_index("c")
    sidx = lax.axis_index("s")
    gid = cidx * NS + sidx
    base = gid * TROWS

    pltpu.sync_copy(decayed_hbm.at[pl.ds(base, TROWS)], proto_v)
    pltpu.sync_copy(cid_hbm, ids_v.at[pl.ds(0, B)])
    pltpu.sync_copy(hitcnt_hbm.at[gid], hits_v.at[pl.ds(0, NGRP)])

    def _chunk(ch, carry):
        pltpu.sync_copy(we_hbm.at[pl.ds(ch * WCHUNK, WCHUNK)], we_v)

        def _group(g, c2):
            nh = hits_v[pl.ds(ch * (WCHUNK // 16) + g, 16)][0]

            @pl.when(nh > 0.0)
            def _():
                def _sample(j, c3):
                    i = g * 16 + j
                    loc = ids_v[pl.ds(ch * WCHUNK + i, 16)][0] - base

                    @pl.when((loc >= 0) & (loc < TROWS))
                    def _():
                        proto_v[loc, pl.ds(0, 16)] = (
                            proto_v[loc, pl.ds(0, 16)] + we_v[i, pl.ds(0, 16)])
                        proto_v[loc, pl.ds(16, 16)] = (
                            proto_v[loc, pl.ds(16, 16)] + we_v[i, pl.ds(16, 16)])

                    return c3

                lax.fori_loop(0, 16, _sample, 0)

            return c2

        lax.fori_loop(0, WCHUNK // 16, _group, 0)
        return carry

    lax.fori_loop(0, B // WCHUNK, _chunk, 0)

    pltpu.sync_copy(proto_v, out_hbm.at[pl.ds(base, TROWS)])


def kernel(embeddings, cluster_ids, prototypes):
    cid_col = cluster_ids.reshape(B, 1)
    cid_row = cluster_ids.reshape(1, B)
    we, decayed, hitcnt = _weights_call(cid_col, cid_row, embeddings, prototypes)
    hitcnt_t = hitcnt.transpose(1, 0, 2).reshape(NTILES, NGRP)
    loss2d = _loss_call(embeddings, prototypes, cid_col)
    new_protos = _make_sc_update()(decayed, cluster_ids, we, hitcnt_t)
    return loss2d[0, 0], new_protos


# speedup vs baseline: 1.8835x; 1.0750x over previous
import functools
import math

import jax
import jax.numpy as jnp
from jax import lax
from jax.experimental import pallas as pl
from jax.experimental.pallas import tpu as pltpu
from jax.experimental.pallas import tpu_sc as plsc

B = 4096
P = 8192
D = 32
TEMP = 0.5
MOM = 0.9
LN_M = math.log(MOM)

RB = 8
BR = B // RB
CB = 8
PC = P // CB

NC = 2
NS = 16
NTILES = NC * NS
TROWS = P // NTILES
TSHIFT = 8
WCHUNK = 256
GRP = 8
NGRP = B // GRP


def _weights_body(cidr_ref, cidc_ref, emb_ref, proto_ref,
                  we_ref, decayed_ref, hitcnt_ref):
    r = pl.program_id(0)
    cidr = cidr_ref[...]
    row_gid = r * BR + lax.broadcasted_iota(jnp.int32, (BR, 1), 0)

    def step(k, acc):
        cidc = cidc_ref[:, pl.ds(k * 1024, 1024)]
        col_gid = k * 1024 + lax.broadcasted_iota(jnp.int32, (1, 1024), 1)
        hit = (cidr == cidc) & (col_gid > row_gid)
        return acc + jnp.sum(jnp.where(hit, 1.0, 0.0), axis=1, keepdims=True)

    occ_after = lax.fori_loop(0, B // 1024, step, jnp.zeros((BR, 1), jnp.float32))
    w = (1.0 - MOM) * jnp.exp(occ_after * LN_M)
    we_ref[...] = emb_ref[...] * w

    colv = r * PC + lax.broadcasted_iota(jnp.int32, (PC, 1), 0)

    def cstep(k, acc):
        cidc = cidc_ref[:, pl.ds(k * 1024, 1024)]
        return acc + jnp.sum(jnp.where(colv == cidc, 1.0, 0.0),
                             axis=1, keepdims=True)

    cnt = lax.fori_loop(0, B // 1024, cstep, jnp.zeros((PC, 1), jnp.float32))
    decayed_ref[...] = proto_ref[...] * jnp.exp(cnt * LN_M)

    cids = cidc_ref[:, pl.ds(r * BR, BR)]
    tilev = lax.broadcasted_iota(jnp.int32, (NTILES, 1), 0)
    teq = jnp.where(tilev == lax.shift_right_logical(cids, TSHIFT), 1.0, 0.0)
    rowv = lax.broadcasted_iota(jnp.int32, (BR, 1), 0) // GRP
    gsel = jnp.where(rowv == lax.broadcasted_iota(jnp.int32, (1, BR // GRP), 1),
                     1.0, 0.0)
    hc = lax.dot_general(teq, gsel, (((1,), (0,)), ((), ())),
                         preferred_element_type=jnp.float32)
    hitcnt_ref[...] = hc.reshape(1, NTILES, BR // GRP)


def _loss_body(emb_ref, proto_ref, cidcol_ref,
               loss_ref,
               sumexp_ref, pos_ref, lossacc_ref):
    r = pl.program_id(0)
    c = pl.program_id(1)

    emb = emb_ref[...]
    en = emb * lax.rsqrt(jnp.maximum(jnp.sum(emb * emb, axis=1, keepdims=True), 1e-24))
    pr = proto_ref[...]
    pn = pr * lax.rsqrt(jnp.maximum(jnp.sum(pr * pr, axis=1, keepdims=True), 1e-24))
    s = lax.dot_general(en, pn, (((1,), (1,)), ((), ())),
                        preferred_element_type=jnp.float32) * (1.0 / TEMP)

    cid_col = cidcol_ref[...]
    col_gid = c * PC + lax.broadcasted_iota(jnp.int32, (1, PC), 1)
    is_pos = cid_col == col_gid

    prev_se = jnp.where(c == 0, jnp.zeros((BR, 1), jnp.float32), sumexp_ref[...])
    sumexp_ref[...] = prev_se + jnp.sum(jnp.exp(s), axis=1, keepdims=True)
    prev_pos = jnp.where(c == 0, jnp.zeros((BR, 1), jnp.float32), pos_ref[...])
    pos_ref[...] = prev_pos + jnp.sum(jnp.where(is_pos, s, 0.0), axis=1, keepdims=True)

    @pl.when(c == CB - 1)
    def _finish_row_block():
        pos = pos_ref[...]
        se = sumexp_ref[...]
        row_loss = -pos + jnp.log(se - jnp.exp(pos))
        prev = jnp.where(r == 0, jnp.zeros((1, 1), jnp.float32), lossacc_ref[...])
        lossacc_ref[...] = prev + jnp.sum(row_loss, axis=(0, 1), keepdims=True)

    @pl.when((c == CB - 1) & (r == RB - 1))
    def _emit_loss():
        loss_ref[...] = lossacc_ref[...] * (1.0 / B)


_weights_call = pl.pallas_call(
    _weights_body,
    grid=(RB,),
    in_specs=[
        pl.BlockSpec((BR, 1), lambda r: (r, 0)),
        pl.BlockSpec((1, B), lambda r: (0, 0)),
        pl.BlockSpec((BR, D), lambda r: (r, 0)),
        pl.BlockSpec((PC, D), lambda r: (r, 0)),
    ],
    out_specs=[
        pl.BlockSpec((BR, D), lambda r: (r, 0)),
        pl.BlockSpec((PC, D), lambda r: (r, 0)),
        pl.BlockSpec((1, NTILES, BR // GRP), lambda r: (r, 0, 0)),
    ],
    out_shape=[
        jax.ShapeDtypeStruct((B, D), jnp.float32),
        jax.ShapeDtypeStruct((P, D), jnp.float32),
        jax.ShapeDtypeStruct((RB, NTILES, BR // GRP), jnp.float32),
    ],
)

_loss_call = pl.pallas_call(
    _loss_body,
    grid=(RB, CB),
    in_specs=[
        pl.BlockSpec((BR, D), lambda r, c: (r, 0)),
        pl.BlockSpec((PC, D), lambda r, c: (c, 0)),
        pl.BlockSpec((BR, 1), lambda r, c: (r, 0)),
    ],
    out_specs=pl.BlockSpec((1, 1), lambda r, c: (0, 0)),
    out_shape=jax.ShapeDtypeStruct((1, 1), jnp.float32),
    scratch_shapes=[
        pltpu.VMEM((BR, 1), jnp.float32),
        pltpu.VMEM((BR, 1), jnp.float32),
        pltpu.VMEM((1, 1), jnp.float32),
    ],
)


@functools.cache
def _make_sc_update():
    return pl.kernel(
        _sc_update_body,
        out_type=jax.ShapeDtypeStruct((P, D), jnp.float32),
        mesh=plsc.VectorSubcoreMesh(core_axis_name="c", subcore_axis_name="s",
                                    num_cores=NC, num_subcores=NS),
        scratch_types=[
            pltpu.VMEM((TROWS, D), jnp.float32),
            pltpu.VMEM((B + 16,), jnp.int32),
            pltpu.VMEM((WCHUNK, D), jnp.float32),
            pltpu.VMEM((NGRP + 16,), jnp.float32),
        ],
    )


def _sc_update_body(decayed_hbm, cid_hbm, we_hbm, hitcnt_hbm, out_hbm,
                    proto_v, ids_v, we_v, hits_v):
    cidx = lax.axis_index("c")
    sidx = lax.axis_index("s")
    gid = cidx * NS + sidx
    base = gid * TROWS

    pltpu.sync_copy(decayed_hbm.at[pl.ds(base, TROWS)], proto_v)
    pltpu.sync_copy(cid_hbm, ids_v.at[pl.ds(0, B)])
    pltpu.sync_copy(hitcnt_hbm.at[gid], hits_v.at[pl.ds(0, NGRP)])

    def _chunk(ch, carry):
        pltpu.sync_copy(we_hbm.at[pl.ds(ch * WCHUNK, WCHUNK)], we_v)

        def _group(g, c2):
            nh = hits_v[pl.ds(ch * (WCHUNK // GRP) + g, 16)][0]

            @pl.when(nh > 0.0)
            def _():
                for j in range(GRP):
                    i = g * GRP + j
                    loc = ids_v[pl.ds(ch * WCHUNK + i, 16)][0] - base

                    @pl.when((loc >= 0) & (loc < TROWS))
                    def _():
                        proto_v[loc, pl.ds(0, 16)] = (
                            proto_v[loc, pl.ds(0, 16)] + we_v[i, pl.ds(0, 16)])
                        proto_v[loc, pl.ds(16, 16)] = (
                            proto_v[loc, pl.ds(16, 16)] + we_v[i, pl.ds(16, 16)])

            return c2

        lax.fori_loop(0, WCHUNK // GRP, _group, 0)
        return carry

    lax.fori_loop(0, B // WCHUNK, _chunk, 0)

    pltpu.sync_copy(proto_v, out_hbm.at[pl.ds(base, TROWS)])


def kernel(embeddings, cluster_ids, prototypes):
    cid_col = cluster_ids.reshape(B, 1)
    cid_row = cluster_ids.reshape(1, B)
    we, decayed, hitcnt = _weights_call(cid_col, cid_row, embeddings, prototypes)
    hitcnt_t = hitcnt.transpose(1, 0, 2).reshape(NTILES, NGRP)
    loss2d = _loss_call(embeddings, prototypes, cid_col)
    new_protos = _make_sc_update()(decayed, cluster_ids, we, hitcnt_t)
    return loss2d[0, 0], new_protos
